# baseline (speedup 1.0000x reference)
import functools

import jax
import jax.numpy as jnp
import numpy as np
from jax.experimental import pallas as pl
from jax.experimental.pallas import tpu as pltpu
from jax.experimental.pallas import tpu_sc as plsc

_K_LIST = [20, 11, 7, 6, 5]
_NB = 256
_BIG = 1e30
_BIG2 = 2e30
_BN_S = float(1.0 / np.sqrt(1.0 + 1e-5))


_CW = 512


def _knn_body(k, n, xb_ref, xt_ref, bcol_ref, brow_ref, lo_ref, nc_ref,
              idx_ref, d2_ref):
    i = pl.program_id(0)
    lo = lo_ref[i]
    nc = nc_ref[i]
    nb = xb_ref.shape[0]
    cw = _CW
    cmax = jnp.int32(n // cw - 1)
    xb = xb_ref[...]
    sq_col = jnp.sum(xb * xb, axis=1, keepdims=True)
    bc = bcol_ref[...]

    def _cs(c):
        return jnp.minimum(lo + c, cmax) * cw

    def compute_chunk(c, carry):
        cs = _cs(c)
        xt_c = xt_ref[:, pl.ds(cs, cw)]
        prod = jnp.dot(xb, xt_c, preferred_element_type=jnp.float32)
        sq_row = jnp.sum(xt_c * xt_c, axis=0, keepdims=True)
        d2c = sq_col + sq_row - 2.0 * prod
        mask = bc != brow_ref[:, pl.ds(cs, cw)]
        d2_ref[:, pl.ds(cs, cw)] = jnp.where(mask, jnp.float32(_BIG), d2c)
        return carry

    jax.lax.fori_loop(0, nc, compute_chunk, 0)

    col_local = jax.lax.broadcasted_iota(jnp.int32, (nb, cw), 1)
    sel = jnp.full((nb, 1), -1, jnp.int32)
    for t in range(k):
        def scan_chunk(c, carry):
            acc_v, acc_i = carry
            cs = _cs(c)
            colg = col_local + cs
            blk = d2_ref[:, pl.ds(cs, cw)]
            blk = jnp.where(colg == sel, jnp.float32(_BIG2), blk)
            d2_ref[:, pl.ds(cs, cw)] = blk
            cm = jnp.min(blk, axis=1, keepdims=True)
            ci = jnp.min(jnp.where(blk <= cm, colg, jnp.int32(n)), axis=1,
                         keepdims=True)
            upd = (cm < acc_v) | ((cm == acc_v) & (ci < acc_i))
            return (jnp.where(upd, cm, acc_v), jnp.where(upd, ci, acc_i))

        _, sel = jax.lax.fori_loop(
            0, nc, scan_chunk,
            (jnp.full((nb, 1), jnp.inf, jnp.float32),
             jnp.full((nb, 1), n, jnp.int32)),
        )
        idx_ref[:, t : t + 1] = sel


def _knn(x, xt, bcol, brow, lo, nc, k):
    n, d = x.shape
    nb = _NB
    return pl.pallas_call(
        functools.partial(_knn_body, k, n),
        grid=(n // nb,),
        in_specs=[
            pl.BlockSpec((nb, d), lambda i: (i, 0)),
            pl.BlockSpec((d, n), lambda i: (0, 0)),
            pl.BlockSpec((nb, 1), lambda i: (i, 0)),
            pl.BlockSpec((1, n), lambda i: (0, 0)),
            pl.BlockSpec(memory_space=pltpu.MemorySpace.SMEM),
            pl.BlockSpec(memory_space=pltpu.MemorySpace.SMEM),
        ],
        out_specs=pl.BlockSpec((nb, k), lambda i: (i, 0)),
        out_shape=jax.ShapeDtypeStruct((n, k), jnp.int32),
        scratch_shapes=[pltpu.VMEM((nb, n), jnp.float32)],
    )(x, xt, bcol, brow, lo, nc)


def _linear_body(xb_ref, w_ref, b_ref, o_ref):
    o_ref[...] = (
        jnp.dot(xb_ref[...], w_ref[...], preferred_element_type=jnp.float32) + b_ref[...]
    )


def _linear(x, w, b):
    n, d = x.shape
    h = w.shape[1]
    nb = _NB
    return pl.pallas_call(
        _linear_body,
        grid=(n // nb,),
        in_specs=[
            pl.BlockSpec((nb, d), lambda i: (i, 0)),
            pl.BlockSpec((d, h), lambda i: (0, 0)),
            pl.BlockSpec((1, h), lambda i: (0, 0)),
        ],
        out_specs=pl.BlockSpec((nb, h), lambda i: (i, 0)),
        out_shape=jax.ShapeDtypeStruct((n, h), jnp.float32),
    )(x, w, b)


def _sc_gather(data, idx_flat):
    h = data.shape[1]
    kn = idx_flat.shape[1]
    win = 128
    mesh = plsc.VectorSubcoreMesh(core_axis_name="c", subcore_axis_name="s")

    @pl.kernel(out_type=jax.ShapeDtypeStruct((kn, h), data.dtype), mesh=mesh)
    def gather_kernel(d_hbm, i_hbm, o_hbm):
        def body(i_vmem, o_vmem):
            pltpu.sync_copy(d_hbm.at[i_vmem.at[0]], o_vmem)

        pltpu.emit_pipeline(
            body,
            grid=(kn // win,),
            in_specs=[pl.BlockSpec((1, win), lambda i: (0, i))],
            out_specs=[pl.BlockSpec((win, h), lambda i: (i, 0))],
            core_axis_name=("c", "s"),
            dimension_semantics=(pltpu.PARALLEL,),
        )(i_hbm, o_hbm)

    return gather_kernel(data, idx_flat)


def _edge_body(k, has_skip, *refs):
    if has_skip:
        (g_ref, x_ref, w0_ref, b0_ref, bng_ref, bnb_ref, w1_ref, b1_ref,
         w2_ref, b2_ref, g2_ref, sb2_ref, skip_ref, out_ref) = refs
    else:
        (g_ref, x_ref, w0_ref, b0_ref, bng_ref, bnb_ref, w1_ref, b1_ref,
         w2_ref, b2_ref, g2_ref, sb2_ref, out_ref) = refs
    j = pl.program_id(1)
    c = jnp.sqrt(jnp.float32(1.0 + 1e-5))
    xi = x_ref[...]
    xj = g_ref[...]
    e = jnp.concatenate([xi, xj - xi], axis=1)
    h0 = jnp.maximum(
        jnp.dot(e, w0_ref[...], preferred_element_type=jnp.float32) + b0_ref[...], 0.0
    )
    h0 = h0 / c * bng_ref[...] + bnb_ref[...]
    h1 = jnp.maximum(
        jnp.dot(h0, w1_ref[...], preferred_element_type=jnp.float32) + b1_ref[...], 0.0
    )
    h2 = jnp.dot(h1, w2_ref[...], preferred_element_type=jnp.float32) + b2_ref[...]

    @pl.when(j == 0)
    def _():
        out_ref[...] = h2

    @pl.when(j > 0)
    def _():
        out_ref[...] = jnp.maximum(out_ref[...], h2)

    @pl.when(j == k - 1)
    def _():
        res = jnp.maximum(out_ref[...], 0.0) / c * g2_ref[...] + sb2_ref[...]
        if has_skip:
            res = res + skip_ref[...]
        out_ref[...] = res


def _edge(g, xp, w0, b0, bng, bnb, w1, b1, w2, b2, g2, sb2, skip, k):
    n, dp = xp.shape
    h = w1.shape[1]
    nb = _NB
    nblocks = n // nb
    in_specs = [
        pl.BlockSpec((nb, dp), lambda i, j: (j * nblocks + i, 0)),
        pl.BlockSpec((nb, dp), lambda i, j: (i, 0)),
        pl.BlockSpec((2 * dp, h), lambda i, j: (0, 0)),
        pl.BlockSpec((1, h), lambda i, j: (0, 0)),
        pl.BlockSpec((1, h), lambda i, j: (0, 0)),
        pl.BlockSpec((1, h), lambda i, j: (0, 0)),
        pl.BlockSpec((h, h), lambda i, j: (0, 0)),
        pl.BlockSpec((1, h), lambda i, j: (0, 0)),
        pl.BlockSpec((h, h), lambda i, j: (0, 0)),
        pl.BlockSpec((1, h), lambda i, j: (0, 0)),
        pl.BlockSpec((1, h), lambda i, j: (0, 0)),
        pl.BlockSpec((1, h), lambda i, j: (0, 0)),
    ]
    args = [g, xp, w0, b0, bng, bnb, w1, b1, w2, b2, g2, sb2]
    if skip is not None:
        in_specs.append(pl.BlockSpec((nb, h), lambda i, j: (i, 0)))
        args.append(skip)
    return pl.pallas_call(
        functools.partial(_edge_body, k, skip is not None),
        grid=(nblocks, k),
        in_specs=in_specs,
        out_specs=pl.BlockSpec((nb, h), lambda i, j: (i, 0)),
        out_shape=jax.ShapeDtypeStruct((n, h), jnp.float32),
    )(*args)


def _ln(v, g, b):
    m = jnp.mean(v, axis=1, keepdims=True)
    var = jnp.mean((v - m) ** 2, axis=1, keepdims=True)
    return (v - m) / jnp.sqrt(var + 1e-5) * g + b


def _pool_head_body(num_graphs, x_ref, bcol_ref, brow_ref,
                    w0_ref, b0_ref, ln0g_ref, ln0b_ref,
                    w1_ref, b1_ref, ln1g_ref, ln1b_ref,
                    w2_ref, b2_ref, ln2g_ref, ln2b_ref,
                    sk1w_ref, sk1b_ref, sk2w_ref, sk2b_ref,
                    ow_ref, ob_ref, out_ref, emb_ref):
    x = x_ref[...]
    bc = bcol_ref[...]
    del brow_ref
    zero = jnp.zeros_like(x)
    means, maxes, cnts = [], [], []
    for gi in range(num_graphs):
        m = bc == jnp.float32(gi)
        cg = jnp.sum(jnp.where(m, 1.0, 0.0), axis=0, keepdims=True)
        sg = jnp.sum(jnp.where(m, x, zero), axis=0, keepdims=True)
        means.append(sg / cg)
        maxes.append(jnp.max(jnp.where(m, x, jnp.float32(-_BIG)), axis=0,
                             keepdims=True))
        cnts.append(cg)
    mpn = zero
    for gi in range(num_graphs):
        mpn = jnp.where(bc == jnp.float32(gi), means[gi], mpn)
    diff = x - mpn
    d2 = diff * diff
    stds = []
    for gi in range(num_graphs):
        ssg = jnp.sum(jnp.where(bc == jnp.float32(gi), d2, zero), axis=0,
                      keepdims=True)
        stds.append(jnp.sqrt(ssg / jnp.maximum(cnts[gi] - 1.0, 1.0)))
    emb = jnp.concatenate(
        [jnp.concatenate(means, axis=0), jnp.concatenate(maxes, axis=0),
         jnp.concatenate(stds, axis=0)], axis=1)
    emb_ref[...] = emb

    h0 = jnp.dot(emb, w0_ref[...], preferred_element_type=jnp.float32) + b0_ref[...]
    h0 = jnp.maximum(_ln(h0, ln0g_ref[...], ln0b_ref[...]), 0.0)
    h1 = jnp.dot(h0, w1_ref[...], preferred_element_type=jnp.float32) + b1_ref[...]
    h1 = jnp.maximum(_ln(h1, ln1g_ref[...], ln1b_ref[...]), 0.0)
    h1 = h1 + jnp.dot(emb, sk1w_ref[...], preferred_element_type=jnp.float32) + sk1b_ref[...]
    h2 = jnp.dot(h1, w2_ref[...], preferred_element_type=jnp.float32) + b2_ref[...]
    h2 = jnp.maximum(_ln(h2, ln2g_ref[...], ln2b_ref[...]), 0.0)
    h2 = h2 + jnp.dot(emb, sk2w_ref[...], preferred_element_type=jnp.float32) + sk2b_ref[...]
    out_ref[...] = jnp.dot(h2, ow_ref[...], preferred_element_type=jnp.float32) + ob_ref[...]


def _pool_head(x, bcol, brow, hp, num_graphs):
    n, h = x.shape
    row = lambda v: v.reshape(1, -1)
    args = (
        x, bcol, brow,
        hp["l0"]["W"], row(hp["l0"]["b"]), row(hp["ln0_g"]), row(hp["ln0_b"]),
        hp["l1"]["W"], row(hp["l1"]["b"]), row(hp["ln1_g"]), row(hp["ln1_b"]),
        hp["l2"]["W"], row(hp["l2"]["b"]), row(hp["ln2_g"]), row(hp["ln2_b"]),
        hp["sk1"]["W"], row(hp["sk1"]["b"]), hp["sk2"]["W"], row(hp["sk2"]["b"]),
        hp["out"]["W"], row(hp["out"]["b"]),
    )
    return pl.pallas_call(
        functools.partial(_pool_head_body, num_graphs),
        out_shape=[
            jax.ShapeDtypeStruct((num_graphs, 1), jnp.float32),
            jax.ShapeDtypeStruct((num_graphs, 3 * h), jnp.float32),
        ],
    )(*args)


def kernel(x, params, edge_index, batch):
    del edge_index
    n = x.shape[0]
    num_graphs = 4
    bcol = batch.astype(jnp.float32).reshape(n, 1)
    brow = batch.astype(jnp.float32).reshape(1, n)
    nblocks = n // _NB
    b2 = batch.reshape(nblocks, _NB)
    seg_l = jnp.searchsorted(batch, b2[:, 0]).astype(jnp.int32)
    seg_r = jnp.searchsorted(batch, b2[:, -1], side="right").astype(jnp.int32)
    lo = seg_l // _CW
    nc = (seg_r - lo * _CW + _CW - 1) // _CW
    xc = x
    for i in range(5):
        p = params["convs"][i]
        k = _K_LIST[i]
        d = xc.shape[1]
        h = p["l0"]["W"].shape[1]
        dp = ((d + 127) // 128) * 128
        xp = jnp.pad(xc, ((0, 0), (0, dp - d)))
        w0 = p["l0"]["W"]
        w0p = jnp.zeros((2 * dp, h), jnp.float32)
        w0p = w0p.at[:d].set(w0[:d]).at[dp : dp + d].set(w0[d:])
        b0 = p["l0"]["b"].reshape(1, h)
        bng = p["bn_g"].reshape(1, h)
        bnb = p["bn_b"].reshape(1, h)
        w1 = p["l1"]["W"]
        b1 = p["l1"]["b"].reshape(1, h)
        w2 = p["l2"]["W"]
        b2 = p["l2"]["b"].reshape(1, h)
        g2 = params["bns"][i]["g"].reshape(1, h)
        sb2 = params["bns"][i]["b"].reshape(1, h)
        idx = _knn(xp, xp.T, bcol, brow, lo, nc, k)
        g = _sc_gather(xp, idx.T.reshape(1, k * n))
        skip = None
        if i == 4:
            skip = _linear(xc, params["skip3"]["W"], params["skip3"]["b"].reshape(1, -1))
        xc = _edge(g, xp, w0p, b0, bng, bnb, w1, b1, w2, b2, g2, sb2, skip, k)
    out, emb = _pool_head(xc, bcol, brow, params["head"], num_graphs)
    return out, emb


# speedup vs baseline: 1.0430x; 1.0430x over previous
import functools

import jax
import jax.numpy as jnp
import numpy as np
from jax.experimental import pallas as pl
from jax.experimental.pallas import tpu as pltpu
from jax.experimental.pallas import tpu_sc as plsc

_K_LIST = [20, 11, 7, 6, 5]
_NB = 256
_BIG = 1e30
_BIG2 = 2e30
_BN_S = float(1.0 / np.sqrt(1.0 + 1e-5))


_CW = 512


def _knn_body(k, n, xb_ref, xt_ref, bcol_ref, brow_ref, lo_ref, nc_ref,
              idx_ref, d2_ref):
    i = pl.program_id(0)
    lo = lo_ref[i]
    nc = nc_ref[i]
    nb = xb_ref.shape[0]
    cw = _CW
    cmax = jnp.int32(n // cw - 1)
    xb = xb_ref[...]
    sq_col = jnp.sum(xb * xb, axis=1, keepdims=True)
    bc = bcol_ref[...]

    def _cs(c):
        return jnp.minimum(lo + c, cmax) * cw

    def compute_chunk(c, carry):
        cs = _cs(c)
        xt_c = xt_ref[:, pl.ds(cs, cw)]
        prod = jnp.dot(xb, xt_c, preferred_element_type=jnp.float32)
        sq_row = jnp.sum(xt_c * xt_c, axis=0, keepdims=True)
        d2c = sq_col + sq_row - 2.0 * prod
        mask = bc != brow_ref[:, pl.ds(cs, cw)]
        d2_ref[:, pl.ds(cs, cw)] = jnp.where(mask, jnp.float32(_BIG), d2c)
        return carry

    jax.lax.fori_loop(0, nc, compute_chunk, 0)

    col_local = jax.lax.broadcasted_iota(jnp.int32, (nb, cw), 1)
    sel = jnp.full((nb, 1), -1, jnp.int32)
    for t in range(k):
        def scan_chunk(c, carry):
            acc_v, acc_i = carry
            cs = _cs(c)
            colg = col_local + cs
            blk = d2_ref[:, pl.ds(cs, cw)]
            blk = jnp.where(colg == sel, jnp.float32(_BIG2), blk)
            d2_ref[:, pl.ds(cs, cw)] = blk
            cm = jnp.min(blk, axis=1, keepdims=True)
            ci = jnp.min(jnp.where(blk <= cm, colg, jnp.int32(n)), axis=1,
                         keepdims=True)
            upd = (cm < acc_v) | ((cm == acc_v) & (ci < acc_i))
            return (jnp.where(upd, cm, acc_v), jnp.where(upd, ci, acc_i))

        _, sel = jax.lax.fori_loop(
            0, nc, scan_chunk,
            (jnp.full((nb, 1), jnp.inf, jnp.float32),
             jnp.full((nb, 1), n, jnp.int32)),
        )
        idx_ref[:, t : t + 1] = sel


def _knn(x, xt, bcol, brow, lo, nc, k, off_b, nrows):
    n, d = x.shape
    nb = _NB
    return pl.pallas_call(
        functools.partial(_knn_body, k, n),
        grid=(nrows // nb,),
        in_specs=[
            pl.BlockSpec((nb, d), lambda i: (i + off_b, 0)),
            pl.BlockSpec((d, n), lambda i: (0, 0)),
            pl.BlockSpec((nb, 1), lambda i: (i + off_b, 0)),
            pl.BlockSpec((1, n), lambda i: (0, 0)),
            pl.BlockSpec(memory_space=pltpu.MemorySpace.SMEM),
            pl.BlockSpec(memory_space=pltpu.MemorySpace.SMEM),
        ],
        out_specs=pl.BlockSpec((nb, k), lambda i: (i, 0)),
        out_shape=jax.ShapeDtypeStruct((nrows, k), jnp.int32),
        scratch_shapes=[pltpu.VMEM((nb, n), jnp.float32)],
    )(x, xt, bcol, brow, lo, nc)


def _linear_body(xb_ref, w_ref, b_ref, o_ref):
    o_ref[...] = (
        jnp.dot(xb_ref[...], w_ref[...], preferred_element_type=jnp.float32) + b_ref[...]
    )


def _linear(x, w, b):
    n, d = x.shape
    h = w.shape[1]
    nb = _NB
    return pl.pallas_call(
        _linear_body,
        grid=(n // nb,),
        in_specs=[
            pl.BlockSpec((nb, d), lambda i: (i, 0)),
            pl.BlockSpec((d, h), lambda i: (0, 0)),
            pl.BlockSpec((1, h), lambda i: (0, 0)),
        ],
        out_specs=pl.BlockSpec((nb, h), lambda i: (i, 0)),
        out_shape=jax.ShapeDtypeStruct((n, h), jnp.float32),
    )(x, w, b)


def _sc_gather(data, idx_flat, win=128):
    h = data.shape[1]
    kn = idx_flat.shape[1]
    mesh = plsc.VectorSubcoreMesh(core_axis_name="c", subcore_axis_name="s")

    @pl.kernel(out_type=jax.ShapeDtypeStruct((kn, h), data.dtype), mesh=mesh)
    def gather_kernel(d_hbm, i_hbm, o_hbm):
        def body(i_vmem, o_vmem):
            pltpu.sync_copy(d_hbm.at[i_vmem.at[0]], o_vmem)

        pltpu.emit_pipeline(
            body,
            grid=(kn // win,),
            in_specs=[pl.BlockSpec((1, win), lambda i: (0, i))],
            out_specs=[pl.BlockSpec((win, h), lambda i: (i, 0))],
            core_axis_name=("c", "s"),
            dimension_semantics=(pltpu.PARALLEL,),
        )(i_hbm, o_hbm)

    return gather_kernel(data, idx_flat)


def _edge_body(k, has_skip, *refs):
    if has_skip:
        (g_ref, x_ref, w0_ref, b0_ref, bng_ref, bnb_ref, w1_ref, b1_ref,
         w2_ref, b2_ref, g2_ref, sb2_ref, skip_ref, out_ref) = refs
    else:
        (g_ref, x_ref, w0_ref, b0_ref, bng_ref, bnb_ref, w1_ref, b1_ref,
         w2_ref, b2_ref, g2_ref, sb2_ref, out_ref) = refs
    j = pl.program_id(1)
    c = jnp.sqrt(jnp.float32(1.0 + 1e-5))
    xi = x_ref[...]
    xj = g_ref[...]
    e = jnp.concatenate([xi, xj - xi], axis=1)
    h0 = jnp.maximum(
        jnp.dot(e, w0_ref[...], preferred_element_type=jnp.float32) + b0_ref[...], 0.0
    )
    h0 = h0 / c * bng_ref[...] + bnb_ref[...]
    h1 = jnp.maximum(
        jnp.dot(h0, w1_ref[...], preferred_element_type=jnp.float32) + b1_ref[...], 0.0
    )
    h2 = jnp.dot(h1, w2_ref[...], preferred_element_type=jnp.float32) + b2_ref[...]

    @pl.when(j == 0)
    def _():
        out_ref[...] = h2

    @pl.when(j > 0)
    def _():
        out_ref[...] = jnp.maximum(out_ref[...], h2)

    @pl.when(j == k - 1)
    def _():
        res = jnp.maximum(out_ref[...], 0.0) / c * g2_ref[...] + sb2_ref[...]
        if has_skip:
            res = res + skip_ref[...]
        out_ref[...] = res


def _edge(g, xp, w0, b0, bng, bnb, w1, b1, w2, b2, g2, sb2, skip, k, off_b, nrows):
    n, dp = xp.shape
    h = w1.shape[1]
    nb = _NB
    nblocks = nrows // nb
    in_specs = [
        pl.BlockSpec((nb, dp), lambda i, j: (j * nblocks + i, 0)),
        pl.BlockSpec((nb, dp), lambda i, j: (i + off_b, 0)),
        pl.BlockSpec((2 * dp, h), lambda i, j: (0, 0)),
        pl.BlockSpec((1, h), lambda i, j: (0, 0)),
        pl.BlockSpec((1, h), lambda i, j: (0, 0)),
        pl.BlockSpec((1, h), lambda i, j: (0, 0)),
        pl.BlockSpec((h, h), lambda i, j: (0, 0)),
        pl.BlockSpec((1, h), lambda i, j: (0, 0)),
        pl.BlockSpec((h, h), lambda i, j: (0, 0)),
        pl.BlockSpec((1, h), lambda i, j: (0, 0)),
        pl.BlockSpec((1, h), lambda i, j: (0, 0)),
        pl.BlockSpec((1, h), lambda i, j: (0, 0)),
    ]
    args = [g, xp, w0, b0, bng, bnb, w1, b1, w2, b2, g2, sb2]
    if skip is not None:
        in_specs.append(pl.BlockSpec((nb, h), lambda i, j: (i + off_b, 0)))
        args.append(skip)
    return pl.pallas_call(
        functools.partial(_edge_body, k, skip is not None),
        grid=(nblocks, k),
        in_specs=in_specs,
        out_specs=pl.BlockSpec((nb, h), lambda i, j: (i, 0)),
        out_shape=jax.ShapeDtypeStruct((nrows, h), jnp.float32),
    )(*args)


def _ln(v, g, b):
    m = jnp.mean(v, axis=1, keepdims=True)
    var = jnp.mean((v - m) ** 2, axis=1, keepdims=True)
    return (v - m) / jnp.sqrt(var + 1e-5) * g + b


def _pool_head_body(num_graphs, x_ref, bcol_ref, brow_ref,
                    w0_ref, b0_ref, ln0g_ref, ln0b_ref,
                    w1_ref, b1_ref, ln1g_ref, ln1b_ref,
                    w2_ref, b2_ref, ln2g_ref, ln2b_ref,
                    sk1w_ref, sk1b_ref, sk2w_ref, sk2b_ref,
                    ow_ref, ob_ref, out_ref, emb_ref):
    x = x_ref[...]
    bc = bcol_ref[...]
    del brow_ref
    zero = jnp.zeros_like(x)
    means, maxes, cnts = [], [], []
    for gi in range(num_graphs):
        m = bc == jnp.float32(gi)
        cg = jnp.sum(jnp.where(m, 1.0, 0.0), axis=0, keepdims=True)
        sg = jnp.sum(jnp.where(m, x, zero), axis=0, keepdims=True)
        means.append(sg / cg)
        maxes.append(jnp.max(jnp.where(m, x, jnp.float32(-_BIG)), axis=0,
                             keepdims=True))
        cnts.append(cg)
    mpn = zero
    for gi in range(num_graphs):
        mpn = jnp.where(bc == jnp.float32(gi), means[gi], mpn)
    diff = x - mpn
    d2 = diff * diff
    stds = []
    for gi in range(num_graphs):
        ssg = jnp.sum(jnp.where(bc == jnp.float32(gi), d2, zero), axis=0,
                      keepdims=True)
        stds.append(jnp.sqrt(ssg / jnp.maximum(cnts[gi] - 1.0, 1.0)))
    emb = jnp.concatenate(
        [jnp.concatenate(means, axis=0), jnp.concatenate(maxes, axis=0),
         jnp.concatenate(stds, axis=0)], axis=1)
    emb_ref[...] = emb

    h0 = jnp.dot(emb, w0_ref[...], preferred_element_type=jnp.float32) + b0_ref[...]
    h0 = jnp.maximum(_ln(h0, ln0g_ref[...], ln0b_ref[...]), 0.0)
    h1 = jnp.dot(h0, w1_ref[...], preferred_element_type=jnp.float32) + b1_ref[...]
    h1 = jnp.maximum(_ln(h1, ln1g_ref[...], ln1b_ref[...]), 0.0)
    h1 = h1 + jnp.dot(emb, sk1w_ref[...], preferred_element_type=jnp.float32) + sk1b_ref[...]
    h2 = jnp.dot(h1, w2_ref[...], preferred_element_type=jnp.float32) + b2_ref[...]
    h2 = jnp.maximum(_ln(h2, ln2g_ref[...], ln2b_ref[...]), 0.0)
    h2 = h2 + jnp.dot(emb, sk2w_ref[...], preferred_element_type=jnp.float32) + sk2b_ref[...]
    out_ref[...] = jnp.dot(h2, ow_ref[...], preferred_element_type=jnp.float32) + ob_ref[...]


def _pool_head(x, bcol, brow, hp, num_graphs):
    n, h = x.shape
    row = lambda v: v.reshape(1, -1)
    args = (
        x, bcol, brow,
        hp["l0"]["W"], row(hp["l0"]["b"]), row(hp["ln0_g"]), row(hp["ln0_b"]),
        hp["l1"]["W"], row(hp["l1"]["b"]), row(hp["ln1_g"]), row(hp["ln1_b"]),
        hp["l2"]["W"], row(hp["l2"]["b"]), row(hp["ln2_g"]), row(hp["ln2_b"]),
        hp["sk1"]["W"], row(hp["sk1"]["b"]), hp["sk2"]["W"], row(hp["sk2"]["b"]),
        hp["out"]["W"], row(hp["out"]["b"]),
    )
    return pl.pallas_call(
        functools.partial(_pool_head_body, num_graphs),
        out_shape=[
            jax.ShapeDtypeStruct((num_graphs, 1), jnp.float32),
            jax.ShapeDtypeStruct((num_graphs, 3 * h), jnp.float32),
        ],
    )(*args)


def kernel(x, params, edge_index, batch):
    del edge_index
    n = x.shape[0]
    num_graphs = 4
    bcol = batch.astype(jnp.float32).reshape(n, 1)
    brow = batch.astype(jnp.float32).reshape(1, n)
    nblocks = n // _NB
    b2 = batch.reshape(nblocks, _NB)
    seg_l = jnp.searchsorted(batch, b2[:, 0]).astype(jnp.int32)
    seg_r = jnp.searchsorted(batch, b2[:, -1], side="right").astype(jnp.int32)
    lo = seg_l // _CW
    nc = (seg_r - lo * _CW + _CW - 1) // _CW
    xc = x
    for i in range(5):
        p = params["convs"][i]
        k = _K_LIST[i]
        d = xc.shape[1]
        h = p["l0"]["W"].shape[1]
        dp = ((d + 127) // 128) * 128
        xp = jnp.pad(xc, ((0, 0), (0, dp - d)))
        w0 = p["l0"]["W"]
        w0p = jnp.zeros((2 * dp, h), jnp.float32)
        w0p = w0p.at[:d].set(w0[:d]).at[dp : dp + d].set(w0[d:])
        b0 = p["l0"]["b"].reshape(1, h)
        bng = p["bn_g"].reshape(1, h)
        bnb = p["bn_b"].reshape(1, h)
        w1 = p["l1"]["W"]
        b1 = p["l1"]["b"].reshape(1, h)
        w2 = p["l2"]["W"]
        b2 = p["l2"]["b"].reshape(1, h)
        g2 = params["bns"][i]["g"].reshape(1, h)
        sb2 = params["bns"][i]["b"].reshape(1, h)
        skip = None
        if i == 4:
            skip = _linear(xc, params["skip3"]["W"], params["skip3"]["b"].reshape(1, -1))
        xpt = xp.T
        halves = []
        nh = n // 2
        nhb = nh // _NB
        for half in range(2):
            off_b = half * nhb
            idx_h = _knn(xp, xpt, bcol, brow, lo[off_b : off_b + nhb],
                         nc[off_b : off_b + nhb], k, off_b, nh)
            g_h = _sc_gather(xp, idx_h.T.reshape(1, k * nh))
            halves.append(
                _edge(g_h, xp, w0p, b0, bng, bnb, w1, b1, w2, b2, g2, sb2,
                      skip, k, off_b, nh)
            )
        xc = jnp.concatenate(halves, axis=0)
    out, emb = _pool_head(xc, bcol, brow, params["head"], num_graphs)
    return out, emb


# speedup vs baseline: 1.1130x; 1.0671x over previous
import functools

import jax
import jax.numpy as jnp
import numpy as np
from jax.experimental import pallas as pl
from jax.experimental.pallas import tpu as pltpu
from jax.experimental.pallas import tpu_sc as plsc

_K_LIST = [20, 11, 7, 6, 5]
_NB = 256
_BIG = 1e30
_BIG2 = 2e30
_BN_S = float(1.0 / np.sqrt(1.0 + 1e-5))


_CW = 1024


def _knn_body(k, n, xb_ref, xt_ref, bcol_ref, brow_ref, lo_ref, nc_ref,
              idx_ref, d2_ref):
    i = pl.program_id(0)
    lo = lo_ref[i]
    nc = nc_ref[i]
    nb = xb_ref.shape[0]
    cw = _CW
    cmax = jnp.int32(n // cw - 1)
    xb = xb_ref[...]
    sq_col = jnp.sum(xb * xb, axis=1, keepdims=True)
    bc = bcol_ref[...]

    def _cs(c):
        return jnp.minimum(lo + c, cmax) * cw

    def compute_chunk(c, carry):
        cs = _cs(c)
        xt_c = xt_ref[:, pl.ds(cs, cw)]
        prod = jnp.dot(xb, xt_c, preferred_element_type=jnp.float32)
        sq_row = jnp.sum(xt_c * xt_c, axis=0, keepdims=True)
        d2c = sq_col + sq_row - 2.0 * prod
        mask = bc != brow_ref[:, pl.ds(cs, cw)]
        d2_ref[:, pl.ds(cs, cw)] = jnp.where(mask, jnp.float32(_BIG), d2c)
        return carry

    jax.lax.fori_loop(0, nc, compute_chunk, 0)

    col_local = jax.lax.broadcasted_iota(jnp.int32, (nb, cw), 1)
    sel = jnp.full((nb, 1), -1, jnp.int32)
    for t in range(k):
        def scan_chunk(c, carry):
            acc_v, acc_i = carry
            cs = _cs(c)
            colg = col_local + cs
            blk = d2_ref[:, pl.ds(cs, cw)]
            blk = jnp.where(colg == sel, jnp.float32(_BIG2), blk)
            d2_ref[:, pl.ds(cs, cw)] = blk
            cm = jnp.min(blk, axis=1, keepdims=True)
            ci = jnp.min(jnp.where(blk <= cm, colg, jnp.int32(n)), axis=1,
                         keepdims=True)
            upd = (cm < acc_v) | ((cm == acc_v) & (ci < acc_i))
            return (jnp.where(upd, cm, acc_v), jnp.where(upd, ci, acc_i))

        _, sel = jax.lax.fori_loop(
            0, nc, scan_chunk,
            (jnp.full((nb, 1), jnp.inf, jnp.float32),
             jnp.full((nb, 1), n, jnp.int32)),
        )
        idx_ref[:, t : t + 1] = sel


def _knn(x, xt, bcol, brow, lo, nc, k, off_b, nrows):
    n, d = x.shape
    nb = _NB
    return pl.pallas_call(
        functools.partial(_knn_body, k, n),
        grid=(nrows // nb,),
        in_specs=[
            pl.BlockSpec((nb, d), lambda i: (i + off_b, 0)),
            pl.BlockSpec((d, n), lambda i: (0, 0)),
            pl.BlockSpec((nb, 1), lambda i: (i + off_b, 0)),
            pl.BlockSpec((1, n), lambda i: (0, 0)),
            pl.BlockSpec(memory_space=pltpu.MemorySpace.SMEM),
            pl.BlockSpec(memory_space=pltpu.MemorySpace.SMEM),
        ],
        out_specs=pl.BlockSpec((nb, k), lambda i: (i, 0)),
        out_shape=jax.ShapeDtypeStruct((nrows, k), jnp.int32),
        scratch_shapes=[pltpu.VMEM((nb, n), jnp.float32)],
    )(x, xt, bcol, brow, lo, nc)


def _linear_body(xb_ref, w_ref, b_ref, o_ref):
    o_ref[...] = (
        jnp.dot(xb_ref[...], w_ref[...], preferred_element_type=jnp.float32) + b_ref[...]
    )


def _linear(x, w, b):
    n, d = x.shape
    h = w.shape[1]
    nb = _NB
    return pl.pallas_call(
        _linear_body,
        grid=(n // nb,),
        in_specs=[
            pl.BlockSpec((nb, d), lambda i: (i, 0)),
            pl.BlockSpec((d, h), lambda i: (0, 0)),
            pl.BlockSpec((1, h), lambda i: (0, 0)),
        ],
        out_specs=pl.BlockSpec((nb, h), lambda i: (i, 0)),
        out_shape=jax.ShapeDtypeStruct((n, h), jnp.float32),
    )(x, w, b)


def _sc_gather(data, idx_flat, win=128):
    h = data.shape[1]
    kn = idx_flat.shape[1]
    mesh = plsc.VectorSubcoreMesh(core_axis_name="c", subcore_axis_name="s")

    @pl.kernel(out_type=jax.ShapeDtypeStruct((kn, h), data.dtype), mesh=mesh)
    def gather_kernel(d_hbm, i_hbm, o_hbm):
        def body(i_vmem, o_vmem):
            pltpu.sync_copy(d_hbm.at[i_vmem.at[0]], o_vmem)

        pltpu.emit_pipeline(
            body,
            grid=(kn // win,),
            in_specs=[pl.BlockSpec((1, win), lambda i: (0, i))],
            out_specs=[pl.BlockSpec((win, h), lambda i: (i, 0))],
            core_axis_name=("c", "s"),
            dimension_semantics=(pltpu.PARALLEL,),
        )(i_hbm, o_hbm)

    return gather_kernel(data, idx_flat)


def _edge_body(k, has_skip, *refs):
    if has_skip:
        (g_ref, x_ref, w0_ref, b0_ref, bng_ref, bnb_ref, w1_ref, b1_ref,
         w2_ref, b2_ref, g2_ref, sb2_ref, skip_ref, out_ref) = refs
    else:
        (g_ref, x_ref, w0_ref, b0_ref, bng_ref, bnb_ref, w1_ref, b1_ref,
         w2_ref, b2_ref, g2_ref, sb2_ref, out_ref) = refs
    j = pl.program_id(1)
    c = jnp.sqrt(jnp.float32(1.0 + 1e-5))
    xi = x_ref[...]
    xj = g_ref[...]
    e = jnp.concatenate([xi, xj - xi], axis=1)
    h0 = jnp.maximum(
        jnp.dot(e, w0_ref[...], preferred_element_type=jnp.float32) + b0_ref[...], 0.0
    )
    h0 = h0 / c * bng_ref[...] + bnb_ref[...]
    h1 = jnp.maximum(
        jnp.dot(h0, w1_ref[...], preferred_element_type=jnp.float32) + b1_ref[...], 0.0
    )
    h2 = jnp.dot(h1, w2_ref[...], preferred_element_type=jnp.float32) + b2_ref[...]

    @pl.when(j == 0)
    def _():
        out_ref[...] = h2

    @pl.when(j > 0)
    def _():
        out_ref[...] = jnp.maximum(out_ref[...], h2)

    @pl.when(j == k - 1)
    def _():
        res = jnp.maximum(out_ref[...], 0.0) / c * g2_ref[...] + sb2_ref[...]
        if has_skip:
            res = res + skip_ref[...]
        out_ref[...] = res


def _edge(g, xp, w0, b0, bng, bnb, w1, b1, w2, b2, g2, sb2, skip, k, off_b, nrows):
    n, dp = xp.shape
    h = w1.shape[1]
    nb = _NB
    nblocks = nrows // nb
    in_specs = [
        pl.BlockSpec((nb, dp), lambda i, j: (j * nblocks + i, 0)),
        pl.BlockSpec((nb, dp), lambda i, j: (i + off_b, 0)),
        pl.BlockSpec((2 * dp, h), lambda i, j: (0, 0)),
        pl.BlockSpec((1, h), lambda i, j: (0, 0)),
        pl.BlockSpec((1, h), lambda i, j: (0, 0)),
        pl.BlockSpec((1, h), lambda i, j: (0, 0)),
        pl.BlockSpec((h, h), lambda i, j: (0, 0)),
        pl.BlockSpec((1, h), lambda i, j: (0, 0)),
        pl.BlockSpec((h, h), lambda i, j: (0, 0)),
        pl.BlockSpec((1, h), lambda i, j: (0, 0)),
        pl.BlockSpec((1, h), lambda i, j: (0, 0)),
        pl.BlockSpec((1, h), lambda i, j: (0, 0)),
    ]
    args = [g, xp, w0, b0, bng, bnb, w1, b1, w2, b2, g2, sb2]
    if skip is not None:
        in_specs.append(pl.BlockSpec((nb, h), lambda i, j: (i + off_b, 0)))
        args.append(skip)
    return pl.pallas_call(
        functools.partial(_edge_body, k, skip is not None),
        grid=(nblocks, k),
        in_specs=in_specs,
        out_specs=pl.BlockSpec((nb, h), lambda i, j: (i, 0)),
        out_shape=jax.ShapeDtypeStruct((nrows, h), jnp.float32),
    )(*args)


def _ln(v, g, b):
    m = jnp.mean(v, axis=1, keepdims=True)
    var = jnp.mean((v - m) ** 2, axis=1, keepdims=True)
    return (v - m) / jnp.sqrt(var + 1e-5) * g + b


def _pool_head_body(num_graphs, x_ref, bcol_ref, brow_ref,
                    w0_ref, b0_ref, ln0g_ref, ln0b_ref,
                    w1_ref, b1_ref, ln1g_ref, ln1b_ref,
                    w2_ref, b2_ref, ln2g_ref, ln2b_ref,
                    sk1w_ref, sk1b_ref, sk2w_ref, sk2b_ref,
                    ow_ref, ob_ref, out_ref, emb_ref):
    x = x_ref[...]
    bc = bcol_ref[...]
    del brow_ref
    zero = jnp.zeros_like(x)
    means, maxes, cnts = [], [], []
    for gi in range(num_graphs):
        m = bc == jnp.float32(gi)
        cg = jnp.sum(jnp.where(m, 1.0, 0.0), axis=0, keepdims=True)
        sg = jnp.sum(jnp.where(m, x, zero), axis=0, keepdims=True)
        means.append(sg / cg)
        maxes.append(jnp.max(jnp.where(m, x, jnp.float32(-_BIG)), axis=0,
                             keepdims=True))
        cnts.append(cg)
    mpn = zero
    for gi in range(num_graphs):
        mpn = jnp.where(bc == jnp.float32(gi), means[gi], mpn)
    diff = x - mpn
    d2 = diff * diff
    stds = []
    for gi in range(num_graphs):
        ssg = jnp.sum(jnp.where(bc == jnp.float32(gi), d2, zero), axis=0,
                      keepdims=True)
        stds.append(jnp.sqrt(ssg / jnp.maximum(cnts[gi] - 1.0, 1.0)))
    emb = jnp.concatenate(
        [jnp.concatenate(means, axis=0), jnp.concatenate(maxes, axis=0),
         jnp.concatenate(stds, axis=0)], axis=1)
    emb_ref[...] = emb

    h0 = jnp.dot(emb, w0_ref[...], preferred_element_type=jnp.float32) + b0_ref[...]
    h0 = jnp.maximum(_ln(h0, ln0g_ref[...], ln0b_ref[...]), 0.0)
    h1 = jnp.dot(h0, w1_ref[...], preferred_element_type=jnp.float32) + b1_ref[...]
    h1 = jnp.maximum(_ln(h1, ln1g_ref[...], ln1b_ref[...]), 0.0)
    h1 = h1 + jnp.dot(emb, sk1w_ref[...], preferred_element_type=jnp.float32) + sk1b_ref[...]
    h2 = jnp.dot(h1, w2_ref[...], preferred_element_type=jnp.float32) + b2_ref[...]
    h2 = jnp.maximum(_ln(h2, ln2g_ref[...], ln2b_ref[...]), 0.0)
    h2 = h2 + jnp.dot(emb, sk2w_ref[...], preferred_element_type=jnp.float32) + sk2b_ref[...]
    out_ref[...] = jnp.dot(h2, ow_ref[...], preferred_element_type=jnp.float32) + ob_ref[...]


def _pool_head(x, bcol, brow, hp, num_graphs):
    n, h = x.shape
    row = lambda v: v.reshape(1, -1)
    args = (
        x, bcol, brow,
        hp["l0"]["W"], row(hp["l0"]["b"]), row(hp["ln0_g"]), row(hp["ln0_b"]),
        hp["l1"]["W"], row(hp["l1"]["b"]), row(hp["ln1_g"]), row(hp["ln1_b"]),
        hp["l2"]["W"], row(hp["l2"]["b"]), row(hp["ln2_g"]), row(hp["ln2_b"]),
        hp["sk1"]["W"], row(hp["sk1"]["b"]), hp["sk2"]["W"], row(hp["sk2"]["b"]),
        hp["out"]["W"], row(hp["out"]["b"]),
    )
    return pl.pallas_call(
        functools.partial(_pool_head_body, num_graphs),
        out_shape=[
            jax.ShapeDtypeStruct((num_graphs, 1), jnp.float32),
            jax.ShapeDtypeStruct((num_graphs, 3 * h), jnp.float32),
        ],
    )(*args)


def kernel(x, params, edge_index, batch):
    del edge_index
    n = x.shape[0]
    num_graphs = 4
    bcol = batch.astype(jnp.float32).reshape(n, 1)
    brow = batch.astype(jnp.float32).reshape(1, n)
    nblocks = n // _NB
    b2 = batch.reshape(nblocks, _NB)
    seg_l = jnp.searchsorted(batch, b2[:, 0]).astype(jnp.int32)
    seg_r = jnp.searchsorted(batch, b2[:, -1], side="right").astype(jnp.int32)
    lo = seg_l // _CW
    nc = (seg_r - lo * _CW + _CW - 1) // _CW
    xc = x
    for i in range(5):
        p = params["convs"][i]
        k = _K_LIST[i]
        d = xc.shape[1]
        h = p["l0"]["W"].shape[1]
        dp = ((d + 127) // 128) * 128
        xp = jnp.pad(xc, ((0, 0), (0, dp - d)))
        w0 = p["l0"]["W"]
        w0p = jnp.zeros((2 * dp, h), jnp.float32)
        w0p = w0p.at[:d].set(w0[:d]).at[dp : dp + d].set(w0[d:])
        b0 = p["l0"]["b"].reshape(1, h)
        bng = p["bn_g"].reshape(1, h)
        bnb = p["bn_b"].reshape(1, h)
        w1 = p["l1"]["W"]
        b1 = p["l1"]["b"].reshape(1, h)
        w2 = p["l2"]["W"]
        b2 = p["l2"]["b"].reshape(1, h)
        g2 = params["bns"][i]["g"].reshape(1, h)
        sb2 = params["bns"][i]["b"].reshape(1, h)
        skip = None
        if i == 4:
            skip = _linear(xc, params["skip3"]["W"], params["skip3"]["b"].reshape(1, -1))
        xpt = xp.T
        halves = []
        nh = n // 2
        nhb = nh // _NB
        for half in range(2):
            off_b = half * nhb
            idx_h = _knn(xp, xpt, bcol, brow, lo[off_b : off_b + nhb],
                         nc[off_b : off_b + nhb], k, off_b, nh)
            g_h = _sc_gather(xp, idx_h.T.reshape(1, k * nh))
            halves.append(
                _edge(g_h, xp, w0p, b0, bng, bnb, w1, b1, w2, b2, g2, sb2,
                      skip, k, off_b, nh)
            )
        xc = jnp.concatenate(halves, axis=0)
    out, emb = _pool_head(xc, bcol, brow, params["head"], num_graphs)
    return out, emb
